# Initial kernel scaffold; baseline (speedup 1.0000x reference)
import jax
import jax.numpy as jnp
from jax.experimental import pallas as pl


def kernel(x):
    raise NotImplementedError("write your pallas kernel here")


# speedup vs baseline: 3.5578x; 3.5578x over previous
import jax
import jax.numpy as jnp
from jax.experimental import pallas as pl


def _copy_body(x_ref, o_ref):
    B = x_ref.shape[0]
    for b in range(B):
        o_ref[:, b, :] = x_ref[b]


def _transpose_pallas(x):
    B, L, d = x.shape
    TT = 64
    grid = (L // TT,)
    return pl.pallas_call(
        _copy_body,
        grid=grid,
        in_specs=[pl.BlockSpec((B, TT, d), lambda t: (0, t, 0))],
        out_specs=pl.BlockSpec((TT, B, d), lambda t: (t, 0, 0)),
        out_shape=jax.ShapeDtypeStruct((L, B, d), x.dtype),
    )(x)


def kernel(x):
    B, L, d = x.shape
    data = _transpose_pallas(x).reshape(L * B, d)
    batch_sizes = jnp.full((L,), B, dtype=jnp.int64)
    sorted_indices = jnp.arange(B, dtype=jnp.int64)
    unsorted_indices = jnp.arange(B, dtype=jnp.int64)
    return data, batch_sizes, sorted_indices, unsorted_indices
